# baseline (speedup 1.0000x reference)
import functools

import jax
import jax.numpy as jnp
from jax import lax
from jax.experimental import pallas as pl
from jax.experimental.pallas import tpu as pltpu
from jax.experimental.pallas import tpu_sc as plsc

VOCAB = 1000000
EMBED_DIM = 64
BATCH = 4096
SEQ = 200

NB = BATCH * SEQ
NW = 32
ROWS_PER_W = NB // NW
CHUNK = 640
NIDX = CHUNK * 4
ITERS = ROWS_PER_W // CHUNK
PAIRS = ITERS // 2

_mesh = plsc.VectorSubcoreMesh(core_axis_name="c", subcore_axis_name="s")


@functools.partial(
    pl.kernel,
    mesh=_mesh,
    compiler_params=pltpu.CompilerParams(use_tc_tiling_on_sc=False),
    out_type=jax.ShapeDtypeStruct((NB * 4, 16), jnp.float32),
    scratch_types=[
        pltpu.VMEM((2, NIDX), jnp.int32),
        pltpu.VMEM((NIDX, 16), jnp.float32),
        pltpu.VMEM((NIDX, 16), jnp.float32),
        pltpu.SemaphoreType.DMA,
        pltpu.SemaphoreType.DMA,
        pltpu.SemaphoreType.DMA,
        pltpu.SemaphoreType.DMA,
        pltpu.SemaphoreType.DMA,
    ],
)
def _sc_gather(idx_hbm, table_hbm, out_hbm, idx_v, rows0, rows1,
               isem, gsem0, gsem1, ssem0, ssem1):
    wid = lax.axis_index("s") * 2 + lax.axis_index("c")
    base = wid * ROWS_PER_W
    ibase = base * 4

    def load_idx(it, b):
        ioff = pl.multiple_of(ibase + it * NIDX, NIDX)
        pltpu.async_copy(idx_hbm.at[pl.ds(ioff, NIDX)], idx_v.at[b], isem)

    def wait_idx(b):
        pltpu.make_async_copy(
            idx_hbm.at[pl.ds(0, NIDX)], idx_v.at[b], isem).wait()

    def fire_gathers(b, rows, gsem):
        pltpu.async_copy(table_hbm.at[idx_v.at[b]], rows, gsem)

    def drain_gathers(rows, gsem):
        pltpu.make_async_copy(
            table_hbm.at[pl.ds(0, NIDX)], rows, gsem).wait()

    def fire_store(rows, it, ssem):
        off = pl.multiple_of(ibase + it * NIDX, NIDX)
        pltpu.async_copy(rows, out_hbm.at[pl.ds(off, NIDX)], ssem)

    def drain_store(rows, ssem):
        pltpu.make_async_copy(
            rows, out_hbm.at[pl.ds(0, NIDX)], ssem).wait()

    load_idx(0, 0)
    load_idx(1, 1)
    wait_idx(0)
    fire_gathers(0, rows0, gsem0)
    wait_idx(1)
    fire_gathers(1, rows1, gsem1)
    drain_gathers(rows0, gsem0)
    fire_store(rows0, 0, ssem0)

    def pair_body(k, carry):
        it0 = 2 * k
        drain_store(rows0, ssem0)
        load_idx(it0, 0)
        wait_idx(0)
        fire_gathers(0, rows0, gsem0)
        drain_gathers(rows1, gsem1)
        fire_store(rows1, it0 - 1, ssem1)
        drain_store(rows1, ssem1)
        load_idx(it0 + 1, 1)
        wait_idx(1)
        fire_gathers(1, rows1, gsem1)
        drain_gathers(rows0, gsem0)
        fire_store(rows0, it0, ssem0)
        return carry

    lax.fori_loop(1, PAIRS, pair_body, 0)

    drain_gathers(rows1, gsem1)
    fire_store(rows1, ITERS - 1, ssem1)
    drain_store(rows0, ssem0)
    drain_store(rows1, ssem1)


def kernel(sentence, table):
    idx4 = (sentence.astype(jnp.int32).reshape(NB, 1) * 4
            + jnp.arange(4, dtype=jnp.int32)).reshape(NB * 4)
    out = _sc_gather(idx4, table.reshape(VOCAB * 4, 16))
    return out.reshape(BATCH, SEQ, EMBED_DIM)
